# baseline (speedup 1.0000x reference)
import functools

import numpy as np
import jax
import jax.numpy as jnp
from jax.experimental import pallas as pl
from jax.experimental.pallas import tpu as pltpu


def _pool_pad_kernel(x_ref, s_ref, o_ref):
    bn, cin, H, W = x_ref.shape
    cout = o_ref.shape[1]
    ev = x_ref[:, :, pl.ds(0, H // 2, 2), :]
    od = x_ref[:, :, pl.ds(1, H // 2, 2), :]
    a = (ev + od).astype(jnp.bfloat16)
    pooled = jax.lax.dot_general(
        a, s_ref[...], (((3,), (0,)), ((), ())),
        preferred_element_type=jnp.float32)
    o_ref[:, :cin] = pooled
    o_ref[:, cin:] = jnp.zeros_like(o_ref[:, cin:])


@functools.partial(jax.jit, static_argnums=(1, 2))
def _shortcut(x_nchw, out_channels, stride):
    N, cin, H, W = x_nchw.shape
    cout = int(out_channels)
    dtype = x_nchw.dtype

    if stride == 1 and cout == cin:
        return x_nchw

    assert stride == 2 and H % 2 == 0 and W % 2 == 0
    Ho, Wo = H // 2, W // 2

    s_np = np.zeros((W, Wo), np.float32)
    s_np[np.arange(W), np.arange(W) // 2] = 0.25
    s_mat = jnp.asarray(s_np, jnp.bfloat16)

    bn = 8
    while N % bn:
        bn //= 2

    return pl.pallas_call(
        _pool_pad_kernel,
        out_shape=jax.ShapeDtypeStruct((N, cout, Ho, Wo), dtype),
        grid=(N // bn,),
        in_specs=[
            pl.BlockSpec((bn, cin, H, W), lambda n: (n, 0, 0, 0)),
            pl.BlockSpec((W, Wo), lambda n: (0, 0)),
        ],
        out_specs=pl.BlockSpec((bn, cout, Ho, Wo), lambda n: (n, 0, 0, 0)),
        compiler_params=pltpu.CompilerParams(
            dimension_semantics=("parallel",)),
        cost_estimate=pl.CostEstimate(
            flops=2 * N * cin * H * W * Wo,
            transcendentals=0,
            bytes_accessed=int((N * cin * H * W + N * cout * Ho * Wo) * 4)),
    )(x_nchw, s_mat)


def kernel(x_nchw):
    return _shortcut(x_nchw, 128, 2)


# speedup vs baseline: 14.1479x; 14.1479x over previous
import functools

import numpy as np
import jax
import jax.numpy as jnp
from jax.experimental import pallas as pl
from jax.experimental.pallas import tpu as pltpu


def _pool_pad_kernel(x_ref, e_ref, o_ref):
    cin, bh, W, N = x_ref.shape
    ho, wo = bh // 2, W // 2
    x00 = x_ref[:, pl.ds(0, ho, 2), pl.ds(0, wo, 2), :]
    x01 = x_ref[:, pl.ds(0, ho, 2), pl.ds(1, wo, 2), :]
    x10 = x_ref[:, pl.ds(1, ho, 2), pl.ds(0, wo, 2), :]
    x11 = x_ref[:, pl.ds(1, ho, 2), pl.ds(1, wo, 2), :]
    s = ((x00 + x01) + (x10 + x11)).astype(jnp.bfloat16)
    t = jax.lax.dot_general(
        s, e_ref[...], (((0,), (0,)), ((), ())),
        preferred_element_type=jnp.float32)
    o_ref[...] = jnp.transpose(t, (2, 0, 1, 3))


@functools.partial(jax.jit, static_argnums=(1, 2))
def _shortcut(x_nchw, out_channels, stride):
    N, cin, H, W = x_nchw.shape
    cout = int(out_channels)
    dtype = x_nchw.dtype

    if stride == 1 and cout == cin:
        return x_nchw

    assert stride == 2 and H % 2 == 0 and W % 2 == 0
    Ho, Wo = H // 2, W // 2

    e_np = np.zeros((cin, cout), np.float32)
    e_np[np.arange(cin), np.arange(cin)] = 0.25
    e_mat = jnp.asarray(e_np, jnp.bfloat16)

    bh = 8
    while H % bh:
        bh //= 2

    x_t = jnp.transpose(x_nchw, (1, 2, 3, 0))
    out_t = pl.pallas_call(
        _pool_pad_kernel,
        out_shape=jax.ShapeDtypeStruct((N, Ho, Wo, cout), dtype),
        grid=(H // bh,),
        in_specs=[
            pl.BlockSpec((cin, bh, W, N), lambda g: (0, g, 0, 0)),
            pl.BlockSpec((cin, cout), lambda g: (0, 0)),
        ],
        out_specs=pl.BlockSpec((N, bh // 2, Wo, cout), lambda g: (0, g, 0, 0)),
        compiler_params=pltpu.CompilerParams(
            dimension_semantics=("parallel",)),
        cost_estimate=pl.CostEstimate(
            flops=2 * N * cin * Ho * Wo * cout,
            transcendentals=0,
            bytes_accessed=int((N * cin * H * W + N * cout * Ho * Wo) * 4)),
    )(x_t, e_mat)
    return jnp.transpose(out_t, (0, 3, 1, 2))


def kernel(x_nchw):
    return _shortcut(x_nchw, 128, 2)
